# baseline (speedup 1.0000x reference)
import functools

import jax
import jax.numpy as jnp
from jax import lax
from jax.experimental import pallas as pl
from jax.experimental.pallas import tpu as pltpu
from jax.experimental.pallas import tpu_sc as plsc

_BOUNDS = (-2.0, -1.5, -1.0, -0.5, 0.0, 0.5, 1.0, 1.5, 2.0)

_ROWS, _COLS = 16384, 26
_NC, _NS, _L = 2, 16, 16
_NW = _NC * _NS
_COLS_W = _ROWS // _NW
_VECS = _COLS_W // _L

_mesh = plsc.VectorSubcoreMesh(core_axis_name="c", subcore_axis_name="s")


@functools.partial(
    pl.kernel,
    mesh=_mesh,
    out_type=jax.ShapeDtypeStruct((_COLS, _ROWS), jnp.int32),
    scratch_types=[
        pltpu.VMEM((_COLS, _COLS_W), jnp.float32),
        pltpu.VMEM((_COLS, _COLS_W), jnp.int32),
    ],
    compiler_params=pltpu.CompilerParams(
        use_tc_tiling_on_sc=True, skip_device_barrier=True),
)
def _bucketize_sc(x_hbm, out_hbm, x_v, o_v):
    wid = lax.axis_index("s") * _NC + lax.axis_index("c")
    c0 = wid * _COLS_W
    pltpu.sync_copy(x_hbm.at[:, pl.ds(c0, _COLS_W)], x_v)

    bvecs = [jnp.full((_L,), b, jnp.float32) for b in _BOUNDS]
    one = jnp.ones((_L,), jnp.int32)
    zero = jnp.zeros((_L,), jnp.int32)

    def bucketize(x):
        acc = zero
        for bv in bvecs:
            acc = acc + jnp.where(x > bv, one, zero)
        return acc

    @plsc.parallel_loop(0, _VECS, step=1)
    def body(v):
        base = v * _L

        def half(h, carry):
            rbase = h * (_COLS // 2)
            for r in range(_COLS // 2):
                row = rbase + r
                o_v[row, pl.ds(base, _L)] = bucketize(x_v[row, pl.ds(base, _L)])
            return carry

        lax.fori_loop(0, 2, half, 0)

    pltpu.sync_copy(o_v, out_hbm.at[:, pl.ds(c0, _COLS_W)])


def kernel(inputs):
    return _bucketize_sc(inputs.T).T


# speedup vs baseline: 1.0586x; 1.0586x over previous
import functools

import jax
import jax.numpy as jnp
from jax import lax
from jax.experimental import pallas as pl
from jax.experimental.pallas import tpu as pltpu
from jax.experimental.pallas import tpu_sc as plsc

_BOUNDS = (-2.0, -1.5, -1.0, -0.5, 0.0, 0.5, 1.0, 1.5, 2.0)

_ROWS, _COLS = 16384, 26
_NC, _NS, _L = 2, 16, 16
_NW = _NC * _NS
_SPLIT = 8192
_TC_COLS = _ROWS - _SPLIT
_COLS_W = _SPLIT // _NW
_VECS = _COLS_W // _L

_mesh = plsc.VectorSubcoreMesh(core_axis_name="c", subcore_axis_name="s")


@functools.partial(
    pl.kernel,
    mesh=_mesh,
    out_type=jax.ShapeDtypeStruct((_COLS, _SPLIT), jnp.int32),
    scratch_types=[
        pltpu.VMEM((_COLS, _COLS_W), jnp.float32),
        pltpu.VMEM((_COLS, _COLS_W), jnp.int32),
    ],
    compiler_params=pltpu.CompilerParams(use_tc_tiling_on_sc=True),
)
def _bucketize_sc(x_hbm, out_hbm, x_v, o_v):
    wid = lax.axis_index("s") * _NC + lax.axis_index("c")
    c0 = wid * _COLS_W
    pltpu.sync_copy(x_hbm.at[:, pl.ds(c0, _COLS_W)], x_v)

    bvecs = [jnp.full((_L,), b, jnp.float32) for b in _BOUNDS]
    one = jnp.ones((_L,), jnp.int32)
    zero = jnp.zeros((_L,), jnp.int32)

    def bucketize(x):
        acc = zero
        for bv in bvecs:
            acc = acc + jnp.where(x > bv, one, zero)
        return acc

    @plsc.parallel_loop(0, _VECS, step=1)
    def body(v):
        base = v * _L
        for r in range(_COLS):
            o_v[r, pl.ds(base, _L)] = bucketize(x_v[r, pl.ds(base, _L)])

    pltpu.sync_copy(o_v, out_hbm.at[:, pl.ds(c0, _COLS_W)])


def _bucketize_tc_body(x_ref, o_ref):
    x = x_ref[...]
    acc = jnp.zeros(x.shape, jnp.int32)
    one = jnp.ones(x.shape, jnp.int32)
    for b in _BOUNDS:
        acc = acc + jnp.where(x > b, one, 0)
    o_ref[...] = acc


_bucketize_tc = pl.pallas_call(
    _bucketize_tc_body,
    out_shape=jax.ShapeDtypeStruct((_COLS, _TC_COLS), jnp.int32),
    grid=(1,),
    in_specs=[pl.BlockSpec((_COLS, _TC_COLS), lambda i: (0, _SPLIT // _TC_COLS))],
    out_specs=pl.BlockSpec((_COLS, _TC_COLS), lambda i: (0, 0)),
)


def kernel(inputs):
    xt = inputs.T
    y_sc = _bucketize_sc(xt)
    y_tc = _bucketize_tc(xt)
    return jnp.concatenate([y_sc, y_tc], axis=1).T
